# baseline (speedup 1.0000x reference)
import jax
import jax.numpy as jnp
from jax import lax
from jax.experimental import pallas as pl
from jax.experimental.pallas import tpu as pltpu
from jax.experimental.pallas import tpu_sc as plsc

N_NODES = 10000
N_EDGES = 320000
D = 128

NC, NS, LANES = 2, 16, 16
NW = NC * NS
NPAD = 10240
ROWS_PER_TILE = NPAD // NS
CH = 128
K = 80
EPW = K * CH
EPAD = NW * EPW
DH = D // NC
KA = EPAD // (NS * CH)
CHG = 256
KG = EPAD // (NS * CHG)
KGQ = KG
KAQ = KA
BM = 512

_MESH = plsc.VectorSubcoreMesh(
    core_axis_name="c", subcore_axis_name="s", num_cores=NC, num_subcores=NS
)


def _deg_body(dst_hbm, out_hbm, dst_loc, deg_loc, red_v, sum_v, part_sh):
    c = lax.axis_index("c")
    s = lax.axis_index("s")
    w = s * NC + c

    zero16 = jnp.zeros((LANES,), jnp.float32)

    def zbody(i, carry):
        deg_loc[pl.ds(i * LANES, LANES)] = zero16
        return carry

    lax.fori_loop(0, NPAD // LANES, zbody, 0)

    pltpu.sync_copy(dst_hbm.at[w], dst_loc)

    ones16 = jnp.ones((LANES,), jnp.float32)

    def hbody(k, carry):
        idx = dst_loc[pl.ds(k * LANES, LANES)]
        plsc.addupdate_scatter(deg_loc, [idx], ones16)
        return carry

    lax.fori_loop(0, EPW // LANES, hbody, 0)

    pltpu.sync_copy(deg_loc, part_sh.at[s])
    plsc.subcore_barrier()

    pltpu.sync_copy(part_sh.at[:, pl.ds(s * ROWS_PER_TILE, ROWS_PER_TILE)], red_v)

    def rbody(j, carry):
        acc = jnp.zeros((LANES,), jnp.float32)
        for r in range(NS):
            acc = acc + red_v[r, pl.ds(j * LANES, LANES)]
        sum_v[pl.ds(j * LANES, LANES)] = acc
        return carry

    lax.fori_loop(0, ROWS_PER_TILE // LANES, rbody, 0)

    pltpu.sync_copy(sum_v, out_hbm.at[c, pl.ds(s * ROWS_PER_TILE, ROWS_PER_TILE)])


_deg_call = pl.kernel(
    _deg_body,
    out_type=jax.ShapeDtypeStruct((NC, NPAD), jnp.float32),
    mesh=_MESH,
    scratch_types=[
        pltpu.VMEM((EPW,), jnp.int32),
        pltpu.VMEM((NPAD,), jnp.float32),
        pltpu.VMEM((NS, ROWS_PER_TILE), jnp.float32),
        pltpu.VMEM((ROWS_PER_TILE,), jnp.float32),
        pltpu.VMEM_SHARED((NS, NPAD), jnp.float32),
    ],
    compiler_params=pltpu.CompilerParams(needs_layout_passes=False),
)


def _agg_body(h_hbm, src_hbm, dst_hbm, out_hbm, idx_src, idx_dst, rows, gsem, h_sh, acc_sh):
    c = lax.axis_index("c")
    s = lax.axis_index("s")
    hc = h_hbm.at[c]

    r0 = s * ROWS_PER_TILE
    pltpu.sync_copy(src_hbm.at[s], idx_src)
    pltpu.sync_copy(dst_hbm.at[s], idx_dst)
    spans = [(0, CHG), (CHG, CHG), (2 * CHG, ROWS_PER_TILE - 2 * CHG)]
    for i, (off, ln) in enumerate(spans):
        pltpu.async_copy(
            hc.at[pl.ds(r0 + off, ln)], rows.at[i % 2, pl.ds(0, ln)], gsem
        )
        if i > 0:
            po, pn = spans[i - 1]
            pb = rows.at[(i - 1) % 2, pl.ds(0, pn)]
            pltpu.make_async_copy(hc.at[pl.ds(r0 + po, pn)], pb, gsem).wait()
            pltpu.sync_copy(pb, h_sh.at[pl.ds(r0 + po, pn)])
            pltpu.sync_copy(pb, acc_sh.at[pl.ds(r0 + po, pn)])
    lo, ln = spans[-1]
    lb = rows.at[0, pl.ds(0, ln)]
    pltpu.make_async_copy(hc.at[pl.ds(r0 + lo, ln)], lb, gsem).wait()
    pltpu.sync_copy(lb, h_sh.at[pl.ds(r0 + lo, ln)])
    pltpu.sync_copy(lb, acc_sh.at[pl.ds(r0 + lo, ln)])
    plsc.subcore_barrier()

    for b in range(2):
        pltpu.async_copy(h_sh.at[idx_src.at[b]], rows.at[b], gsem)

    def gbody(g, carry):
        for b in range(2):
            kg = 2 * g + b
            pltpu.make_async_copy(h_sh.at[idx_src.at[kg]], rows.at[b], gsem).wait()
            for half in range(2):
                k = 2 * kg + half
                pltpu.sync_copy(
                    rows.at[b, pl.ds(half * CH, CH)],
                    acc_sh.at[idx_dst.at[k]],
                    add=True,
                )

            @pl.when(g < KGQ // 2 - 1)
            def _():
                pltpu.async_copy(h_sh.at[idx_src.at[kg + 2]], rows.at[b], gsem)

        return carry

    lax.fori_loop(0, KGQ // 2, gbody, 0)

    plsc.subcore_barrier()

    def owait(i):
        off, ln = spans[i]
        pltpu.make_async_copy(
            rows.at[i % 2, pl.ds(0, ln)], out_hbm.at[c, pl.ds(r0 + off, ln)], gsem
        ).wait()

    for i, (off, ln) in enumerate(spans):
        if i >= 2:
            owait(i - 2)
        bb = rows.at[i % 2, pl.ds(0, ln)]
        pltpu.sync_copy(acc_sh.at[pl.ds(r0 + off, ln)], bb)
        pltpu.async_copy(bb, out_hbm.at[c, pl.ds(r0 + off, ln)], gsem)
    owait(len(spans) - 2)
    owait(len(spans) - 1)


_agg_call = pl.kernel(
    _agg_body,
    out_type=jax.ShapeDtypeStruct((NC, NPAD, DH), jnp.bfloat16),
    mesh=_MESH,
    scratch_types=[
        pltpu.VMEM((KGQ, CHG), jnp.int32),
        pltpu.VMEM((KAQ, CH), jnp.int32),
        pltpu.VMEM((2, CHG, DH), jnp.bfloat16),
        pltpu.SemaphoreType.DMA,
        pltpu.VMEM_SHARED((NPAD, DH), jnp.bfloat16),
        pltpu.VMEM_SHARED((NPAD, DH), jnp.bfloat16),
    ],
    compiler_params=pltpu.CompilerParams(use_tc_tiling_on_sc=False),
)


def _dinv(dp_ref):
    deg = dp_ref[0] + dp_ref[1] + 1.0
    return lax.rsqrt(jnp.maximum(deg, 1e-12))


def _mm1_body(dp_ref, x_ref, w_ref, o_ref):
    dinv = _dinv(dp_ref)
    h = jnp.dot(x_ref[...], w_ref[...], preferred_element_type=jnp.float32)
    h = (h * dinv).astype(jnp.bfloat16)
    o_ref[0] = h[:, :DH]
    o_ref[1] = h[:, DH:]


def _mid_body(dp_ref, p_ref, b1_ref, w2_ref, o_ref):
    dinv = _dinv(dp_ref)
    agg = jnp.concatenate([p_ref[0], p_ref[1]], axis=1).astype(jnp.float32)
    t = jnp.maximum(agg * dinv + b1_ref[...], 0.0)
    h2 = jnp.dot(t, w2_ref[...], preferred_element_type=jnp.float32)
    h2 = (h2 * dinv).astype(jnp.bfloat16)
    o_ref[0] = h2[:, :DH]
    o_ref[1] = h2[:, DH:]


def _fin_body(dp_ref, q_ref, b2_ref, o_ref):
    dinv = _dinv(dp_ref)
    agg = jnp.concatenate([q_ref[0], q_ref[1]], axis=1).astype(jnp.float32)
    o_ref[...] = agg * dinv + b2_ref[...]


_G = NPAD // BM

_mm1_call = pl.pallas_call(
    _mm1_body,
    grid=(_G,),
    in_specs=[
        pl.BlockSpec((NC, BM, 1), lambda i: (0, i, 0)),
        pl.BlockSpec((BM, D), lambda i: (i, 0)),
        pl.BlockSpec((D, D), lambda i: (0, 0)),
    ],
    out_specs=pl.BlockSpec((NC, BM, DH), lambda i: (0, i, 0)),
    out_shape=jax.ShapeDtypeStruct((NC, NPAD, DH), jnp.bfloat16),
)

_mid_call = pl.pallas_call(
    _mid_body,
    grid=(_G,),
    in_specs=[
        pl.BlockSpec((NC, BM, 1), lambda i: (0, i, 0)),
        pl.BlockSpec((NC, BM, DH), lambda i: (0, i, 0)),
        pl.BlockSpec((1, D), lambda i: (0, 0)),
        pl.BlockSpec((D, D), lambda i: (0, 0)),
    ],
    out_specs=pl.BlockSpec((NC, BM, DH), lambda i: (0, i, 0)),
    out_shape=jax.ShapeDtypeStruct((NC, NPAD, DH), jnp.bfloat16),
)

_fin_call = pl.pallas_call(
    _fin_body,
    grid=(_G,),
    in_specs=[
        pl.BlockSpec((NC, BM, 1), lambda i: (0, i, 0)),
        pl.BlockSpec((NC, BM, DH), lambda i: (0, i, 0)),
        pl.BlockSpec((1, D), lambda i: (0, 0)),
    ],
    out_specs=pl.BlockSpec((BM, D), lambda i: (i, 0)),
    out_shape=jax.ShapeDtypeStruct((NPAD, D), jnp.float32),
)


def kernel(x, edge_index, W1, b1, W2, b2):
    src = edge_index[0].astype(jnp.int32)
    dst = edge_index[1].astype(jnp.int32)
    pad = jnp.full((EPAD - N_EDGES,), N_NODES, jnp.int32)
    src3 = jnp.concatenate([src, pad]).reshape(NS, KGQ, CHG)
    dst3 = jnp.concatenate([dst, pad]).reshape(NS, KAQ, CH)
    dst2 = dst3.reshape(NW, EPW)
    xp = jnp.zeros((NPAD, D), jnp.float32).at[:N_NODES].set(x)

    deg_parts = _deg_call(dst2)
    dp3 = deg_parts.reshape(NC, NPAD, 1)
    h1 = _mm1_call(dp3, xp, W1)
    p = _agg_call(h1, src3, dst3)
    h2 = _mid_call(dp3, p, b1.reshape(1, D), W2)
    q = _agg_call(h2, src3, dst3)
    outp = _fin_call(dp3, q, b2.reshape(1, D))
    return outp[:N_NODES]


# speedup vs baseline: 1.0150x; 1.0150x over previous
import jax
import jax.numpy as jnp
from jax import lax
from jax.experimental import pallas as pl
from jax.experimental.pallas import tpu as pltpu
from jax.experimental.pallas import tpu_sc as plsc

N_NODES = 10000
N_EDGES = 320000
D = 128

NC, NS, LANES = 2, 16, 16
NW = NC * NS
NPAD = 10240
ROWS_PER_TILE = NPAD // NS
CH = 128
K = 80
EPW = K * CH
EPAD = NW * EPW
DH = D // NC
KA = EPAD // (NS * CH)
CHG = 512
KG = EPAD // (NS * CHG)
KGQ = KG
KAQ = KA
BM = 512

_MESH = plsc.VectorSubcoreMesh(
    core_axis_name="c", subcore_axis_name="s", num_cores=NC, num_subcores=NS
)


def _deg_body(dst_hbm, out_hbm, dst_loc, deg_loc, red_v, sum_v, part_sh):
    c = lax.axis_index("c")
    s = lax.axis_index("s")
    w = s * NC + c

    zero16 = jnp.zeros((LANES,), jnp.float32)

    def zbody(i, carry):
        deg_loc[pl.ds(i * LANES, LANES)] = zero16
        return carry

    lax.fori_loop(0, NPAD // LANES, zbody, 0)

    pltpu.sync_copy(dst_hbm.at[w], dst_loc)

    ones16 = jnp.ones((LANES,), jnp.float32)

    def hbody(k, carry):
        idx = dst_loc[pl.ds(k * LANES, LANES)]
        plsc.addupdate_scatter(deg_loc, [idx], ones16)
        return carry

    lax.fori_loop(0, EPW // LANES, hbody, 0)

    pltpu.sync_copy(deg_loc, part_sh.at[s])
    plsc.subcore_barrier()

    pltpu.sync_copy(part_sh.at[:, pl.ds(s * ROWS_PER_TILE, ROWS_PER_TILE)], red_v)

    def rbody(j, carry):
        acc = jnp.zeros((LANES,), jnp.float32)
        for r in range(NS):
            acc = acc + red_v[r, pl.ds(j * LANES, LANES)]
        sum_v[pl.ds(j * LANES, LANES)] = acc
        return carry

    lax.fori_loop(0, ROWS_PER_TILE // LANES, rbody, 0)

    pltpu.sync_copy(sum_v, out_hbm.at[c, pl.ds(s * ROWS_PER_TILE, ROWS_PER_TILE)])


_deg_call = pl.kernel(
    _deg_body,
    out_type=jax.ShapeDtypeStruct((NC, NPAD), jnp.float32),
    mesh=_MESH,
    scratch_types=[
        pltpu.VMEM((EPW,), jnp.int32),
        pltpu.VMEM((NPAD,), jnp.float32),
        pltpu.VMEM((NS, ROWS_PER_TILE), jnp.float32),
        pltpu.VMEM((ROWS_PER_TILE,), jnp.float32),
        pltpu.VMEM_SHARED((NS, NPAD), jnp.float32),
    ],
    compiler_params=pltpu.CompilerParams(needs_layout_passes=False),
)


def _agg_body(h_hbm, src_hbm, dst_hbm, out_hbm, idx_src, idx_dst, rows, gsem, h_sh, acc_sh):
    c = lax.axis_index("c")
    s = lax.axis_index("s")
    hc = h_hbm.at[c]

    r0 = s * ROWS_PER_TILE
    pltpu.sync_copy(src_hbm.at[s], idx_src)
    pltpu.sync_copy(dst_hbm.at[s], idx_dst)
    spans = [(0, CHG), (CHG, ROWS_PER_TILE - CHG)]
    for i, (off, ln) in enumerate(spans):
        pltpu.async_copy(
            hc.at[pl.ds(r0 + off, ln)], rows.at[i % 2, pl.ds(0, ln)], gsem
        )
        if i > 0:
            po, pn = spans[i - 1]
            pb = rows.at[(i - 1) % 2, pl.ds(0, pn)]
            pltpu.make_async_copy(hc.at[pl.ds(r0 + po, pn)], pb, gsem).wait()
            pltpu.sync_copy(pb, h_sh.at[pl.ds(r0 + po, pn)])
            pltpu.sync_copy(pb, acc_sh.at[pl.ds(r0 + po, pn)])
    lo, ln = spans[-1]
    lb = rows.at[(len(spans) - 1) % 2, pl.ds(0, ln)]
    pltpu.make_async_copy(hc.at[pl.ds(r0 + lo, ln)], lb, gsem).wait()
    pltpu.sync_copy(lb, h_sh.at[pl.ds(r0 + lo, ln)])
    pltpu.sync_copy(lb, acc_sh.at[pl.ds(r0 + lo, ln)])
    plsc.subcore_barrier()

    for b in range(2):
        pltpu.async_copy(h_sh.at[idx_src.at[b]], rows.at[b], gsem)

    def gbody(g, carry):
        for b in range(2):
            kg = 2 * g + b
            pltpu.make_async_copy(h_sh.at[idx_src.at[kg]], rows.at[b], gsem).wait()
            for half in range(CHG // CH):
                k = (CHG // CH) * kg + half
                pltpu.sync_copy(
                    rows.at[b, pl.ds(half * CH, CH)],
                    acc_sh.at[idx_dst.at[k]],
                    add=True,
                )

            @pl.when(g < KGQ // 2 - 1)
            def _():
                pltpu.async_copy(h_sh.at[idx_src.at[kg + 2]], rows.at[b], gsem)

        return carry

    lax.fori_loop(0, KGQ // 2, gbody, 0)

    plsc.subcore_barrier()

    def owait(i):
        off, ln = spans[i]
        pltpu.make_async_copy(
            rows.at[i % 2, pl.ds(0, ln)], out_hbm.at[c, pl.ds(r0 + off, ln)], gsem
        ).wait()

    for i, (off, ln) in enumerate(spans):
        if i >= 2:
            owait(i - 2)
        bb = rows.at[i % 2, pl.ds(0, ln)]
        pltpu.sync_copy(acc_sh.at[pl.ds(r0 + off, ln)], bb)
        pltpu.async_copy(bb, out_hbm.at[c, pl.ds(r0 + off, ln)], gsem)
    owait(len(spans) - 2)
    owait(len(spans) - 1)


_agg_call = pl.kernel(
    _agg_body,
    out_type=jax.ShapeDtypeStruct((NC, NPAD, DH), jnp.bfloat16),
    mesh=_MESH,
    scratch_types=[
        pltpu.VMEM((KGQ, CHG), jnp.int32),
        pltpu.VMEM((KAQ, CH), jnp.int32),
        pltpu.VMEM((2, CHG, DH), jnp.bfloat16),
        pltpu.SemaphoreType.DMA,
        pltpu.VMEM_SHARED((NPAD, DH), jnp.bfloat16),
        pltpu.VMEM_SHARED((NPAD, DH), jnp.bfloat16),
    ],
    compiler_params=pltpu.CompilerParams(use_tc_tiling_on_sc=False),
)


def _dinv(dp_ref):
    deg = dp_ref[0] + dp_ref[1] + 1.0
    return lax.rsqrt(jnp.maximum(deg, 1e-12))


def _mm1_body(dp_ref, x_ref, w_ref, o_ref):
    dinv = _dinv(dp_ref)
    h = jnp.dot(x_ref[...], w_ref[...], preferred_element_type=jnp.float32)
    h = (h * dinv).astype(jnp.bfloat16)
    o_ref[0] = h[:, :DH]
    o_ref[1] = h[:, DH:]


def _mid_body(dp_ref, p_ref, b1_ref, w2_ref, o_ref):
    dinv = _dinv(dp_ref)
    agg = jnp.concatenate([p_ref[0], p_ref[1]], axis=1).astype(jnp.float32)
    t = jnp.maximum(agg * dinv + b1_ref[...], 0.0)
    h2 = jnp.dot(t, w2_ref[...], preferred_element_type=jnp.float32)
    h2 = (h2 * dinv).astype(jnp.bfloat16)
    o_ref[0] = h2[:, :DH]
    o_ref[1] = h2[:, DH:]


def _fin_body(dp_ref, q_ref, b2_ref, o_ref):
    dinv = _dinv(dp_ref)
    agg = jnp.concatenate([q_ref[0], q_ref[1]], axis=1).astype(jnp.float32)
    o_ref[...] = agg * dinv + b2_ref[...]


_G = NPAD // BM

_mm1_call = pl.pallas_call(
    _mm1_body,
    grid=(_G,),
    in_specs=[
        pl.BlockSpec((NC, BM, 1), lambda i: (0, i, 0)),
        pl.BlockSpec((BM, D), lambda i: (i, 0)),
        pl.BlockSpec((D, D), lambda i: (0, 0)),
    ],
    out_specs=pl.BlockSpec((NC, BM, DH), lambda i: (0, i, 0)),
    out_shape=jax.ShapeDtypeStruct((NC, NPAD, DH), jnp.bfloat16),
)

_mid_call = pl.pallas_call(
    _mid_body,
    grid=(_G,),
    in_specs=[
        pl.BlockSpec((NC, BM, 1), lambda i: (0, i, 0)),
        pl.BlockSpec((NC, BM, DH), lambda i: (0, i, 0)),
        pl.BlockSpec((1, D), lambda i: (0, 0)),
        pl.BlockSpec((D, D), lambda i: (0, 0)),
    ],
    out_specs=pl.BlockSpec((NC, BM, DH), lambda i: (0, i, 0)),
    out_shape=jax.ShapeDtypeStruct((NC, NPAD, DH), jnp.bfloat16),
)

_fin_call = pl.pallas_call(
    _fin_body,
    grid=(_G,),
    in_specs=[
        pl.BlockSpec((NC, BM, 1), lambda i: (0, i, 0)),
        pl.BlockSpec((NC, BM, DH), lambda i: (0, i, 0)),
        pl.BlockSpec((1, D), lambda i: (0, 0)),
    ],
    out_specs=pl.BlockSpec((BM, D), lambda i: (i, 0)),
    out_shape=jax.ShapeDtypeStruct((NPAD, D), jnp.float32),
)


def kernel(x, edge_index, W1, b1, W2, b2):
    src = edge_index[0].astype(jnp.int32)
    dst = edge_index[1].astype(jnp.int32)
    pad = jnp.full((EPAD - N_EDGES,), N_NODES, jnp.int32)
    src3 = jnp.concatenate([src, pad]).reshape(NS, KGQ, CHG)
    dst3 = jnp.concatenate([dst, pad]).reshape(NS, KAQ, CH)
    dst2 = dst3.reshape(NW, EPW)
    xp = jnp.zeros((NPAD, D), jnp.float32).at[:N_NODES].set(x)

    deg_parts = _deg_call(dst2)
    dp3 = deg_parts.reshape(NC, NPAD, 1)
    h1 = _mm1_call(dp3, xp, W1)
    p = _agg_call(h1, src3, dst3)
    h2 = _mid_call(dp3, p, b1.reshape(1, D), W2)
    q = _agg_call(h2, src3, dst3)
    outp = _fin_call(dp3, q, b2.reshape(1, D))
    return outp[:N_NODES]


# speedup vs baseline: 1.0368x; 1.0215x over previous
import jax
import jax.numpy as jnp
from jax import lax
from jax.experimental import pallas as pl
from jax.experimental.pallas import tpu as pltpu
from jax.experimental.pallas import tpu_sc as plsc

N_NODES = 10000
N_EDGES = 320000
D = 128

NC, NS, LANES = 2, 16, 16
NW = NC * NS
NPAD = 10240
ROWS_PER_TILE = NPAD // NS
CH = 128
K = 80
EPW = K * CH
EPAD = NW * EPW
DH = D // NC
KA = EPAD // (NS * CH)
CHG = 512
KG = EPAD // (NS * CHG)
KGQ = KG
CHS = 256
KS = EPAD // (NS * CHS)
BM = 512

_MESH = plsc.VectorSubcoreMesh(
    core_axis_name="c", subcore_axis_name="s", num_cores=NC, num_subcores=NS
)


def _deg_body(dst_hbm, out_hbm, dst_loc, deg_loc, red_v, sum_v, part_sh):
    c = lax.axis_index("c")
    s = lax.axis_index("s")
    w = s * NC + c

    zero16 = jnp.zeros((LANES,), jnp.float32)

    def zbody(i, carry):
        deg_loc[pl.ds(i * LANES, LANES)] = zero16
        return carry

    lax.fori_loop(0, NPAD // LANES, zbody, 0)

    pltpu.sync_copy(dst_hbm.at[w], dst_loc)

    ones16 = jnp.ones((LANES,), jnp.float32)

    def hbody(k, carry):
        idx = dst_loc[pl.ds(k * LANES, LANES)]
        plsc.addupdate_scatter(deg_loc, [idx], ones16)
        return carry

    lax.fori_loop(0, EPW // LANES, hbody, 0)

    pltpu.sync_copy(deg_loc, part_sh.at[s])
    plsc.subcore_barrier()

    pltpu.sync_copy(part_sh.at[:, pl.ds(s * ROWS_PER_TILE, ROWS_PER_TILE)], red_v)

    def rbody(j, carry):
        acc = jnp.zeros((LANES,), jnp.float32)
        for r in range(NS):
            acc = acc + red_v[r, pl.ds(j * LANES, LANES)]
        sum_v[pl.ds(j * LANES, LANES)] = acc
        return carry

    lax.fori_loop(0, ROWS_PER_TILE // LANES, rbody, 0)

    pltpu.sync_copy(sum_v, out_hbm.at[c, pl.ds(s * ROWS_PER_TILE, ROWS_PER_TILE)])


_deg_call = pl.kernel(
    _deg_body,
    out_type=jax.ShapeDtypeStruct((NC, NPAD), jnp.float32),
    mesh=_MESH,
    scratch_types=[
        pltpu.VMEM((EPW,), jnp.int32),
        pltpu.VMEM((NPAD,), jnp.float32),
        pltpu.VMEM((NS, ROWS_PER_TILE), jnp.float32),
        pltpu.VMEM((ROWS_PER_TILE,), jnp.float32),
        pltpu.VMEM_SHARED((NS, NPAD), jnp.float32),
    ],
    compiler_params=pltpu.CompilerParams(needs_layout_passes=False),
)


def _agg_body(h_hbm, src_hbm, dst_hbm, out_hbm, idx_src, idx_dst, rows, gsem, h_sh, acc_sh):
    c = lax.axis_index("c")
    s = lax.axis_index("s")
    hc = h_hbm.at[c]

    r0 = s * ROWS_PER_TILE
    pltpu.sync_copy(src_hbm.at[s], idx_src)
    pltpu.sync_copy(dst_hbm.at[s], idx_dst)
    spans = [(0, CHG), (CHG, ROWS_PER_TILE - CHG)]
    for i, (off, ln) in enumerate(spans):
        pltpu.async_copy(
            hc.at[pl.ds(r0 + off, ln)], rows.at[i % 2, pl.ds(0, ln)], gsem
        )
        if i > 0:
            po, pn = spans[i - 1]
            pb = rows.at[(i - 1) % 2, pl.ds(0, pn)]
            pltpu.make_async_copy(hc.at[pl.ds(r0 + po, pn)], pb, gsem).wait()
            pltpu.sync_copy(pb, h_sh.at[pl.ds(r0 + po, pn)])
            pltpu.sync_copy(pb, acc_sh.at[pl.ds(r0 + po, pn)])
    lo, ln = spans[-1]
    lb = rows.at[(len(spans) - 1) % 2, pl.ds(0, ln)]
    pltpu.make_async_copy(hc.at[pl.ds(r0 + lo, ln)], lb, gsem).wait()
    pltpu.sync_copy(lb, h_sh.at[pl.ds(r0 + lo, ln)])
    pltpu.sync_copy(lb, acc_sh.at[pl.ds(r0 + lo, ln)])
    plsc.subcore_barrier()

    for b in range(2):
        pltpu.async_copy(h_sh.at[idx_src.at[b]], rows.at[b], gsem)

    def gbody(g, carry):
        for b in range(2):
            kg = 2 * g + b
            pltpu.make_async_copy(h_sh.at[idx_src.at[kg]], rows.at[b], gsem).wait()
            for half in range(CHG // CHS):
                k = (CHG // CHS) * kg + half
                pltpu.sync_copy(
                    rows.at[b, pl.ds(half * CHS, CHS)],
                    acc_sh.at[idx_dst.at[k]],
                    add=True,
                )

            @pl.when(g < KGQ // 2 - 1)
            def _():
                pltpu.async_copy(h_sh.at[idx_src.at[kg + 2]], rows.at[b], gsem)

        return carry

    lax.fori_loop(0, KGQ // 2, gbody, 0)

    plsc.subcore_barrier()

    def owait(i):
        off, ln = spans[i]
        pltpu.make_async_copy(
            rows.at[i % 2, pl.ds(0, ln)], out_hbm.at[c, pl.ds(r0 + off, ln)], gsem
        ).wait()

    for i, (off, ln) in enumerate(spans):
        if i >= 2:
            owait(i - 2)
        bb = rows.at[i % 2, pl.ds(0, ln)]
        pltpu.sync_copy(acc_sh.at[pl.ds(r0 + off, ln)], bb)
        pltpu.async_copy(bb, out_hbm.at[c, pl.ds(r0 + off, ln)], gsem)
    owait(len(spans) - 2)
    owait(len(spans) - 1)


_agg_call = pl.kernel(
    _agg_body,
    out_type=jax.ShapeDtypeStruct((NC, NPAD, DH), jnp.bfloat16),
    mesh=_MESH,
    scratch_types=[
        pltpu.VMEM((KGQ, CHG), jnp.int32),
        pltpu.VMEM((KS, CHS), jnp.int32),
        pltpu.VMEM((2, CHG, DH), jnp.bfloat16),
        pltpu.SemaphoreType.DMA,
        pltpu.VMEM_SHARED((NPAD, DH), jnp.bfloat16),
        pltpu.VMEM_SHARED((NPAD, DH), jnp.bfloat16),
    ],
    compiler_params=pltpu.CompilerParams(use_tc_tiling_on_sc=False),
)


def _dinv(dp_ref):
    deg = dp_ref[0] + dp_ref[1] + 1.0
    return lax.rsqrt(jnp.maximum(deg, 1e-12))


def _mm1_body(dp_ref, x_ref, w_ref, o_ref):
    dinv = _dinv(dp_ref)
    h = jnp.dot(x_ref[...], w_ref[...], preferred_element_type=jnp.float32)
    h = (h * dinv).astype(jnp.bfloat16)
    o_ref[0] = h[:, :DH]
    o_ref[1] = h[:, DH:]


def _mid_body(dp_ref, p_ref, b1_ref, w2_ref, o_ref):
    dinv = _dinv(dp_ref)
    agg = jnp.concatenate([p_ref[0], p_ref[1]], axis=1).astype(jnp.float32)
    t = jnp.maximum(agg * dinv + b1_ref[...], 0.0)
    h2 = jnp.dot(t, w2_ref[...], preferred_element_type=jnp.float32)
    h2 = (h2 * dinv).astype(jnp.bfloat16)
    o_ref[0] = h2[:, :DH]
    o_ref[1] = h2[:, DH:]


def _fin_body(dp_ref, q_ref, b2_ref, o_ref):
    dinv = _dinv(dp_ref)
    agg = jnp.concatenate([q_ref[0], q_ref[1]], axis=1).astype(jnp.float32)
    o_ref[...] = agg * dinv + b2_ref[...]


_G = NPAD // BM

_mm1_call = pl.pallas_call(
    _mm1_body,
    grid=(_G,),
    in_specs=[
        pl.BlockSpec((NC, BM, 1), lambda i: (0, i, 0)),
        pl.BlockSpec((BM, D), lambda i: (i, 0)),
        pl.BlockSpec((D, D), lambda i: (0, 0)),
    ],
    out_specs=pl.BlockSpec((NC, BM, DH), lambda i: (0, i, 0)),
    out_shape=jax.ShapeDtypeStruct((NC, NPAD, DH), jnp.bfloat16),
)

_mid_call = pl.pallas_call(
    _mid_body,
    grid=(_G,),
    in_specs=[
        pl.BlockSpec((NC, BM, 1), lambda i: (0, i, 0)),
        pl.BlockSpec((NC, BM, DH), lambda i: (0, i, 0)),
        pl.BlockSpec((1, D), lambda i: (0, 0)),
        pl.BlockSpec((D, D), lambda i: (0, 0)),
    ],
    out_specs=pl.BlockSpec((NC, BM, DH), lambda i: (0, i, 0)),
    out_shape=jax.ShapeDtypeStruct((NC, NPAD, DH), jnp.bfloat16),
)

_fin_call = pl.pallas_call(
    _fin_body,
    grid=(_G,),
    in_specs=[
        pl.BlockSpec((NC, BM, 1), lambda i: (0, i, 0)),
        pl.BlockSpec((NC, BM, DH), lambda i: (0, i, 0)),
        pl.BlockSpec((1, D), lambda i: (0, 0)),
    ],
    out_specs=pl.BlockSpec((BM, D), lambda i: (i, 0)),
    out_shape=jax.ShapeDtypeStruct((NPAD, D), jnp.float32),
)


def kernel(x, edge_index, W1, b1, W2, b2):
    src = edge_index[0].astype(jnp.int32)
    dst = edge_index[1].astype(jnp.int32)
    pad = jnp.full((EPAD - N_EDGES,), N_NODES, jnp.int32)
    src3 = jnp.concatenate([src, pad]).reshape(NS, KGQ, CHG)
    dst3 = jnp.concatenate([dst, pad]).reshape(NS, KS, CHS)
    dst2 = dst3.reshape(NW, EPW)
    xp = jnp.zeros((NPAD, D), jnp.float32).at[:N_NODES].set(x)

    deg_parts = _deg_call(dst2)
    dp3 = deg_parts.reshape(NC, NPAD, 1)
    h1 = _mm1_call(dp3, xp, W1)
    p = _agg_call(h1, src3, dst3)
    h2 = _mid_call(dp3, p, b1.reshape(1, D), W2)
    q = _agg_call(h2, src3, dst3)
    outp = _fin_call(dp3, q, b2.reshape(1, D))
    return outp[:N_NODES]


# speedup vs baseline: 1.0694x; 1.0315x over previous
import jax
import jax.numpy as jnp
from jax import lax
from jax.experimental import pallas as pl
from jax.experimental.pallas import tpu as pltpu
from jax.experimental.pallas import tpu_sc as plsc

N_NODES = 10000
N_EDGES = 320000
D = 128

NC, NS, LANES = 2, 16, 16
NW = NC * NS
NPAD = 10240
ROWS_PER_TILE = NPAD // NS
CH = 128
K = 80
EPW = K * CH
EPAD = NW * EPW
DH = D // NC
KA = EPAD // (NS * CH)
CHG = 512
KG = EPAD // (NS * CHG)
KGQ = KG
CHS = 512
KS = EPAD // (NS * CHS)
BM = 512

_MESH = plsc.VectorSubcoreMesh(
    core_axis_name="c", subcore_axis_name="s", num_cores=NC, num_subcores=NS
)


def _deg_body(dst_hbm, out_hbm, dst_loc, deg_loc, red_v, sum_v, part_sh):
    c = lax.axis_index("c")
    s = lax.axis_index("s")
    w = s * NC + c

    zero16 = jnp.zeros((LANES,), jnp.float32)

    def zbody(i, carry):
        deg_loc[pl.ds(i * LANES, LANES)] = zero16
        return carry

    lax.fori_loop(0, NPAD // LANES, zbody, 0)

    pltpu.sync_copy(dst_hbm.at[w], dst_loc)

    ones16 = jnp.ones((LANES,), jnp.float32)

    def hbody(k, carry):
        idx = dst_loc[pl.ds(k * LANES, LANES)]
        plsc.addupdate_scatter(deg_loc, [idx], ones16)
        return carry

    lax.fori_loop(0, EPW // LANES, hbody, 0)

    pltpu.sync_copy(deg_loc, part_sh.at[s])
    plsc.subcore_barrier()

    pltpu.sync_copy(part_sh.at[:, pl.ds(s * ROWS_PER_TILE, ROWS_PER_TILE)], red_v)

    def rbody(j, carry):
        acc = jnp.zeros((LANES,), jnp.float32)
        for r in range(NS):
            acc = acc + red_v[r, pl.ds(j * LANES, LANES)]
        sum_v[pl.ds(j * LANES, LANES)] = acc
        return carry

    lax.fori_loop(0, ROWS_PER_TILE // LANES, rbody, 0)

    pltpu.sync_copy(sum_v, out_hbm.at[c, pl.ds(s * ROWS_PER_TILE, ROWS_PER_TILE)])


_deg_call = pl.kernel(
    _deg_body,
    out_type=jax.ShapeDtypeStruct((NC, NPAD), jnp.float32),
    mesh=_MESH,
    scratch_types=[
        pltpu.VMEM((EPW,), jnp.int32),
        pltpu.VMEM((NPAD,), jnp.float32),
        pltpu.VMEM((NS, ROWS_PER_TILE), jnp.float32),
        pltpu.VMEM((ROWS_PER_TILE,), jnp.float32),
        pltpu.VMEM_SHARED((NS, NPAD), jnp.float32),
    ],
    compiler_params=pltpu.CompilerParams(needs_layout_passes=False),
)


def _agg_body(h_hbm, src_hbm, dst_hbm, out_hbm, idx_src, idx_dst, rows, gsem, h_sh, acc_sh):
    c = lax.axis_index("c")
    s = lax.axis_index("s")
    hc = h_hbm.at[c]

    r0 = s * ROWS_PER_TILE
    pltpu.sync_copy(src_hbm.at[s], idx_src)
    pltpu.sync_copy(dst_hbm.at[s], idx_dst)
    spans = [(0, CHG), (CHG, ROWS_PER_TILE - CHG)]
    for i, (off, ln) in enumerate(spans):
        pltpu.async_copy(
            hc.at[pl.ds(r0 + off, ln)], rows.at[i % 2, pl.ds(0, ln)], gsem
        )
        if i > 0:
            po, pn = spans[i - 1]
            pb = rows.at[(i - 1) % 2, pl.ds(0, pn)]
            pltpu.make_async_copy(hc.at[pl.ds(r0 + po, pn)], pb, gsem).wait()
            pltpu.sync_copy(pb, h_sh.at[pl.ds(r0 + po, pn)])
            pltpu.sync_copy(pb, acc_sh.at[pl.ds(r0 + po, pn)])
    lo, ln = spans[-1]
    lb = rows.at[(len(spans) - 1) % 2, pl.ds(0, ln)]
    pltpu.make_async_copy(hc.at[pl.ds(r0 + lo, ln)], lb, gsem).wait()
    pltpu.sync_copy(lb, h_sh.at[pl.ds(r0 + lo, ln)])
    pltpu.sync_copy(lb, acc_sh.at[pl.ds(r0 + lo, ln)])
    plsc.subcore_barrier()

    for b in range(2):
        pltpu.async_copy(h_sh.at[idx_src.at[b]], rows.at[b], gsem)

    def gbody(g, carry):
        for b in range(2):
            kg = 2 * g + b
            pltpu.make_async_copy(h_sh.at[idx_src.at[kg]], rows.at[b], gsem).wait()
            for half in range(CHG // CHS):
                k = (CHG // CHS) * kg + half
                pltpu.sync_copy(
                    rows.at[b, pl.ds(half * CHS, CHS)],
                    acc_sh.at[idx_dst.at[k]],
                    add=True,
                )

            @pl.when(g < KGQ // 2 - 1)
            def _():
                pltpu.async_copy(h_sh.at[idx_src.at[kg + 2]], rows.at[b], gsem)

        return carry

    lax.fori_loop(0, KGQ // 2, gbody, 0)

    plsc.subcore_barrier()

    def owait(i):
        off, ln = spans[i]
        pltpu.make_async_copy(
            rows.at[i % 2, pl.ds(0, ln)], out_hbm.at[c, pl.ds(r0 + off, ln)], gsem
        ).wait()

    for i, (off, ln) in enumerate(spans):
        if i >= 2:
            owait(i - 2)
        bb = rows.at[i % 2, pl.ds(0, ln)]
        pltpu.sync_copy(acc_sh.at[pl.ds(r0 + off, ln)], bb)
        pltpu.async_copy(bb, out_hbm.at[c, pl.ds(r0 + off, ln)], gsem)
    owait(len(spans) - 2)
    owait(len(spans) - 1)


_agg_call = pl.kernel(
    _agg_body,
    out_type=jax.ShapeDtypeStruct((NC, NPAD, DH), jnp.bfloat16),
    mesh=_MESH,
    scratch_types=[
        pltpu.VMEM((KGQ, CHG), jnp.int32),
        pltpu.VMEM((KS, CHS), jnp.int32),
        pltpu.VMEM((2, CHG, DH), jnp.bfloat16),
        pltpu.SemaphoreType.DMA,
        pltpu.VMEM_SHARED((NPAD, DH), jnp.bfloat16),
        pltpu.VMEM_SHARED((NPAD, DH), jnp.bfloat16),
    ],
    compiler_params=pltpu.CompilerParams(use_tc_tiling_on_sc=False),
)


def _dinv(dp_ref):
    deg = dp_ref[0] + dp_ref[1] + 1.0
    return lax.rsqrt(jnp.maximum(deg, 1e-12))


def _mm1_body(dp_ref, x_ref, w_ref, o_ref):
    dinv = _dinv(dp_ref)
    h = jnp.dot(x_ref[...], w_ref[...], preferred_element_type=jnp.float32)
    h = (h * dinv).astype(jnp.bfloat16)
    o_ref[0] = h[:, :DH]
    o_ref[1] = h[:, DH:]


def _mid_body(dp_ref, p_ref, b1_ref, w2_ref, o_ref):
    dinv = _dinv(dp_ref)
    agg = jnp.concatenate([p_ref[0], p_ref[1]], axis=1).astype(jnp.float32)
    t = jnp.maximum(agg * dinv + b1_ref[...], 0.0)
    h2 = jnp.dot(t, w2_ref[...], preferred_element_type=jnp.float32)
    h2 = (h2 * dinv).astype(jnp.bfloat16)
    o_ref[0] = h2[:, :DH]
    o_ref[1] = h2[:, DH:]


def _fin_body(dp_ref, q_ref, b2_ref, o_ref):
    dinv = _dinv(dp_ref)
    agg = jnp.concatenate([q_ref[0], q_ref[1]], axis=1).astype(jnp.float32)
    o_ref[...] = agg * dinv + b2_ref[...]


_G = NPAD // BM

_mm1_call = pl.pallas_call(
    _mm1_body,
    grid=(_G,),
    in_specs=[
        pl.BlockSpec((NC, BM, 1), lambda i: (0, i, 0)),
        pl.BlockSpec((BM, D), lambda i: (i, 0)),
        pl.BlockSpec((D, D), lambda i: (0, 0)),
    ],
    out_specs=pl.BlockSpec((NC, BM, DH), lambda i: (0, i, 0)),
    out_shape=jax.ShapeDtypeStruct((NC, NPAD, DH), jnp.bfloat16),
)

_mid_call = pl.pallas_call(
    _mid_body,
    grid=(_G,),
    in_specs=[
        pl.BlockSpec((NC, BM, 1), lambda i: (0, i, 0)),
        pl.BlockSpec((NC, BM, DH), lambda i: (0, i, 0)),
        pl.BlockSpec((1, D), lambda i: (0, 0)),
        pl.BlockSpec((D, D), lambda i: (0, 0)),
    ],
    out_specs=pl.BlockSpec((NC, BM, DH), lambda i: (0, i, 0)),
    out_shape=jax.ShapeDtypeStruct((NC, NPAD, DH), jnp.bfloat16),
)

_fin_call = pl.pallas_call(
    _fin_body,
    grid=(_G,),
    in_specs=[
        pl.BlockSpec((NC, BM, 1), lambda i: (0, i, 0)),
        pl.BlockSpec((NC, BM, DH), lambda i: (0, i, 0)),
        pl.BlockSpec((1, D), lambda i: (0, 0)),
    ],
    out_specs=pl.BlockSpec((BM, D), lambda i: (i, 0)),
    out_shape=jax.ShapeDtypeStruct((NPAD, D), jnp.float32),
)


def kernel(x, edge_index, W1, b1, W2, b2):
    src = edge_index[0].astype(jnp.int32)
    dst = edge_index[1].astype(jnp.int32)
    pad = jnp.full((EPAD - N_EDGES,), N_NODES, jnp.int32)
    src3 = jnp.concatenate([src, pad]).reshape(NS, KGQ, CHG)
    dst3 = jnp.concatenate([dst, pad]).reshape(NS, KS, CHS)
    dst2 = dst3.reshape(NW, EPW)
    xp = jnp.zeros((NPAD, D), jnp.float32).at[:N_NODES].set(x)

    deg_parts = _deg_call(dst2)
    dp3 = deg_parts.reshape(NC, NPAD, 1)
    h1 = _mm1_call(dp3, xp, W1)
    p = _agg_call(h1, src3, dst3)
    h2 = _mid_call(dp3, p, b1.reshape(1, D), W2)
    q = _agg_call(h2, src3, dst3)
    outp = _fin_call(dp3, q, b2.reshape(1, D))
    return outp[:N_NODES]
